# baseline (device time: 17735 ns/iter reference)
import jax
import jax.numpy as jnp
from jax import lax
from jax.experimental import pallas as pl
from jax.experimental.pallas import tpu as pltpu

N_DEV = 8
C = 4


def kernel(x):
    m, n = x.shape
    cw = n // C

    def body(x_ref, out_ref, buf_ref, comm_ref, copy_sems, send_sems, recv_sems):
        my = lax.axis_index("i")

        barrier_sem = pltpu.get_barrier_semaphore()
        for o in range(1, N_DEV):
            pl.semaphore_signal(
                barrier_sem, inc=1,
                device_id=((my + o) % N_DEV,),
                device_id_type=pl.DeviceIdType.MESH,
            )

        copies = []
        for c in range(C):
            cp = pltpu.make_async_copy(
                x_ref.at[:, pl.ds(c * cw, cw)],
                buf_ref.at[c],
                copy_sems.at[c],
            )
            cp.start()
            copies.append(cp)

        rdmas = []
        for c in range(C):
            copies[c].wait()
            part = jnp.max(buf_ref[c], axis=0, keepdims=True)
            comm_ref[0, :, c * cw:(c + 1) * cw] = part
            if c == 0:
                pl.semaphore_wait(barrier_sem, N_DEV - 1)
            for o in range(1, N_DEV):
                rdma = pltpu.make_async_remote_copy(
                    src_ref=comm_ref.at[0, :, pl.ds(c * cw, cw)],
                    dst_ref=comm_ref.at[o, :, pl.ds(c * cw, cw)],
                    send_sem=send_sems.at[c * (N_DEV - 1) + o - 1],
                    recv_sem=recv_sems.at[c * (N_DEV - 1) + o - 1],
                    device_id=((my + o) % N_DEV,),
                    device_id_type=pl.DeviceIdType.MESH,
                )
                rdma.start()
                rdmas.append(rdma)

        for c in range(C):
            for o in range(1, N_DEV):
                rdmas[c * (N_DEV - 1) + o - 1].wait_recv()
            out_ref[:, c * cw:(c + 1) * cw] = jnp.max(
                comm_ref[:, :, c * cw:(c + 1) * cw], axis=0
            )

        for r in rdmas:
            r.wait_send()

    return pl.pallas_call(
        body,
        out_shape=jax.ShapeDtypeStruct((1, n), x.dtype),
        in_specs=[pl.BlockSpec(memory_space=pl.ANY)],
        out_specs=pl.BlockSpec(memory_space=pltpu.VMEM),
        scratch_shapes=[
            pltpu.VMEM((C, m, cw), x.dtype),
            pltpu.VMEM((N_DEV, 1, n), x.dtype),
            pltpu.SemaphoreType.DMA((C,)),
            pltpu.SemaphoreType.DMA((C * (N_DEV - 1),)),
            pltpu.SemaphoreType.DMA((C * (N_DEV - 1),)),
        ],
        compiler_params=pltpu.CompilerParams(collective_id=0),
    )(x)


# device time: 14908 ns/iter; 1.1896x vs baseline; 1.1896x over previous
import jax
import jax.numpy as jnp
from jax import lax
from jax.experimental import pallas as pl
from jax.experimental.pallas import tpu as pltpu

N_DEV = 8
K = 4


def kernel(x):
    m, n = x.shape
    ch = m // K

    def body(x_ref, out_ref, buf_ref, comm_ref, copy_sems, send_sems, recv_sems):
        my = lax.axis_index("i")

        barrier_sem = pltpu.get_barrier_semaphore()
        for o in range(1, N_DEV):
            pl.semaphore_signal(
                barrier_sem, inc=1,
                device_id=((my + o) % N_DEV,),
                device_id_type=pl.DeviceIdType.MESH,
            )

        copies = []
        for k in range(K):
            cp = pltpu.make_async_copy(
                x_ref.at[pl.ds(k * ch, ch), :],
                buf_ref.at[k],
                copy_sems.at[k],
            )
            cp.start()
            copies.append(cp)

        rdmas = []
        for k in range(K):
            copies[k].wait()
            comm_ref[0, k] = jnp.max(buf_ref[k], axis=0, keepdims=True)
            if k == 0:
                pl.semaphore_wait(barrier_sem, N_DEV - 1)
            for o in range(1, N_DEV):
                rdma = pltpu.make_async_remote_copy(
                    src_ref=comm_ref.at[0, k],
                    dst_ref=comm_ref.at[o, k],
                    send_sem=send_sems.at[k * (N_DEV - 1) + o - 1],
                    recv_sem=recv_sems.at[k * (N_DEV - 1) + o - 1],
                    device_id=((my + o) % N_DEV,),
                    device_id_type=pl.DeviceIdType.MESH,
                )
                rdma.start()
                rdmas.append(rdma)

        for r in rdmas:
            r.wait_recv()
        out_ref[...] = jnp.max(comm_ref[...], axis=(0, 1))

        for r in rdmas:
            r.wait_send()

    return pl.pallas_call(
        body,
        out_shape=jax.ShapeDtypeStruct((1, n), x.dtype),
        in_specs=[pl.BlockSpec(memory_space=pl.ANY)],
        out_specs=pl.BlockSpec(memory_space=pltpu.VMEM),
        scratch_shapes=[
            pltpu.VMEM((K, ch, n), x.dtype),
            pltpu.VMEM((N_DEV, K, 1, n), x.dtype),
            pltpu.SemaphoreType.DMA((K,)),
            pltpu.SemaphoreType.DMA((K * (N_DEV - 1),)),
            pltpu.SemaphoreType.DMA((K * (N_DEV - 1),)),
        ],
        compiler_params=pltpu.CompilerParams(collective_id=0),
    )(x)


# device time: 13641 ns/iter; 1.3001x vs baseline; 1.0929x over previous
import jax
import jax.numpy as jnp
from jax import lax
from jax.experimental import pallas as pl
from jax.experimental.pallas import tpu as pltpu

N_DEV = 8


def kernel(x):
    m, n = x.shape

    def body(x_ref, out_ref, comm_ref, send_sems, recv_sems):
        my = lax.axis_index("i")

        barrier_sem = pltpu.get_barrier_semaphore()
        for o in range(1, N_DEV):
            pl.semaphore_signal(
                barrier_sem, inc=1,
                device_id=((my + o) % N_DEV,),
                device_id_type=pl.DeviceIdType.MESH,
            )
        pl.semaphore_wait(barrier_sem, N_DEV - 1)

        local = jnp.max(x_ref[...], axis=0, keepdims=True)
        comm_ref[0, :, :] = local

        rdmas = []
        for o in range(1, N_DEV):
            rdma = pltpu.make_async_remote_copy(
                src_ref=comm_ref.at[0],
                dst_ref=comm_ref.at[o],
                send_sem=send_sems.at[o - 1],
                recv_sem=recv_sems.at[o - 1],
                device_id=((my + o) % N_DEV,),
                device_id_type=pl.DeviceIdType.MESH,
            )
            rdma.start()
            rdmas.append(rdma)

        acc = local
        for o in range(1, N_DEV):
            rdmas[o - 1].wait_recv()
            acc = jnp.maximum(acc, comm_ref[o, :, :])
        out_ref[...] = acc

        for o in range(1, N_DEV):
            rdmas[o - 1].wait_send()

    return pl.pallas_call(
        body,
        out_shape=jax.ShapeDtypeStruct((1, n), x.dtype),
        in_specs=[pl.BlockSpec(memory_space=pltpu.VMEM)],
        out_specs=pl.BlockSpec(memory_space=pltpu.VMEM),
        scratch_shapes=[
            pltpu.VMEM((N_DEV, 1, n), x.dtype),
            pltpu.SemaphoreType.DMA((N_DEV - 1,)),
            pltpu.SemaphoreType.DMA((N_DEV - 1,)),
        ],
        compiler_params=pltpu.CompilerParams(collective_id=0),
    )(x)
